# Initial kernel scaffold; baseline (speedup 1.0000x reference)
import jax
import jax.numpy as jnp
from jax.experimental import pallas as pl


def kernel(relative_position_bias_table, relative_position_index):
    raise NotImplementedError("write your pallas kernel here")


# speedup vs baseline: 5.1177x; 5.1177x over previous
import functools

import jax
import jax.numpy as jnp
from jax import lax
from jax.experimental import pallas as pl
from jax.experimental.pallas import tpu as pltpu
from jax.experimental.pallas import tpu_sc as plsc

NUM_HEADS = 16
T = 256
B = T * T
NC, NS, L = 2, 16, 16
NW = NC * NS
BPW = B // NW
TAB = 961 * NUM_HEADS

_mesh = plsc.VectorSubcoreMesh(core_axis_name="c", subcore_axis_name="s")


@functools.partial(
    pl.kernel,
    mesh=_mesh,
    compiler_params=pltpu.CompilerParams(needs_layout_passes=False),
    out_type=jax.ShapeDtypeStruct((NUM_HEADS, B), jnp.float32),
    scratch_types=[
        pltpu.VMEM((TAB,), jnp.float32),
        pltpu.VMEM((BPW,), jnp.int32),
        pltpu.VMEM((NUM_HEADS, BPW), jnp.float32),
    ],
)
def _gather_bias(tab_hbm, idx_hbm, out_hbm, tabv, idxv, outv):
    wid = lax.axis_index("s") * NC + lax.axis_index("c")
    base = wid * BPW
    pltpu.sync_copy(tab_hbm, tabv)
    pltpu.sync_copy(idx_hbm.at[pl.ds(base, BPW)], idxv)

    def body(g, c):
        s = g * L
        ivec = idxv[pl.ds(s, L)]
        fbase = ivec * NUM_HEADS
        for h in range(NUM_HEADS):
            outv[h, pl.ds(s, L)] = plsc.load_gather(tabv, [fbase + h])
        return c

    lax.fori_loop(0, BPW // L, body, 0)
    pltpu.sync_copy(outv, out_hbm.at[:, pl.ds(base, BPW)])


def kernel(relative_position_bias_table, relative_position_index):
    tab = relative_position_bias_table.reshape(-1)
    idx = relative_position_index.reshape(-1)
    out = _gather_bias(tab, idx)
    return out.reshape(1, NUM_HEADS, T, T)


# speedup vs baseline: 6.5665x; 1.2831x over previous
import functools

import jax
import jax.numpy as jnp
from jax import lax
from jax.experimental import pallas as pl
from jax.experimental.pallas import tpu as pltpu
from jax.experimental.pallas import tpu_sc as plsc

NUM_HEADS = 16
T = 256
B = T * T
NC, NS, L = 2, 16, 16
NW = NC * NS
BPW = B // NW
TAB = 961 * NUM_HEADS

_mesh = plsc.VectorSubcoreMesh(core_axis_name="c", subcore_axis_name="s")


@functools.partial(
    pl.kernel,
    mesh=_mesh,
    compiler_params=pltpu.CompilerParams(needs_layout_passes=False),
    out_type=jax.ShapeDtypeStruct((NUM_HEADS, B), jnp.float32),
    scratch_types=[
        pltpu.VMEM((TAB,), jnp.float32),
        pltpu.VMEM((BPW,), jnp.int32),
        pltpu.VMEM((NUM_HEADS, BPW), jnp.float32),
    ],
)
def _gather_bias(tab_hbm, idx_hbm, out_hbm, tabv, idxv, outv):
    wid = lax.axis_index("s") * NC + lax.axis_index("c")
    base = wid * BPW
    pltpu.sync_copy(tab_hbm, tabv)
    pltpu.sync_copy(idx_hbm.at[pl.ds(base, BPW)], idxv)

    def body(g, c):
        s = g * L
        ivec = idxv[pl.ds(s, L)]
        fbase = ivec * NUM_HEADS
        vals = [plsc.load_gather(tabv, [fbase + h]) for h in range(NUM_HEADS)]
        for h in range(NUM_HEADS):
            outv[h, pl.ds(s, L)] = vals[h]
        return c

    lax.fori_loop(0, BPW // L, body, 0)
    pltpu.sync_copy(outv, out_hbm.at[:, pl.ds(base, BPW)])


def kernel(relative_position_bias_table, relative_position_index):
    tab = relative_position_bias_table.reshape(-1)
    idx = relative_position_index.reshape(-1)
    out = _gather_bias(tab, idx)
    return out.reshape(1, NUM_HEADS, T, T)


# speedup vs baseline: 6.6144x; 1.0073x over previous
import functools

import jax
import jax.numpy as jnp
from jax import lax
from jax.experimental import pallas as pl
from jax.experimental.pallas import tpu as pltpu
from jax.experimental.pallas import tpu_sc as plsc

NUM_HEADS = 16
T = 256
NC, NS, L = 2, 16, 16
NW = NC * NS
RPW = T // NW
GROUPS = RPW * T // L

_mesh = plsc.VectorSubcoreMesh(core_axis_name="c", subcore_axis_name="s")


@functools.partial(
    pl.kernel,
    mesh=_mesh,
    compiler_params=pltpu.CompilerParams(
        needs_layout_passes=False, use_tc_tiling_on_sc=False
    ),
    out_type=jax.ShapeDtypeStruct((1, NUM_HEADS, T, T), jnp.float32),
    scratch_types=[
        pltpu.VMEM((961, NUM_HEADS), jnp.float32),
        pltpu.VMEM((RPW, T), jnp.int32),
        pltpu.VMEM((NUM_HEADS, RPW, T), jnp.float32),
    ],
)
def _gather_bias(tab_hbm, idx_hbm, out_hbm, tabv, idxv, outv):
    wid = lax.axis_index("s") * NC + lax.axis_index("c")
    row0 = wid * RPW
    pltpu.sync_copy(tab_hbm, tabv)
    pltpu.sync_copy(idx_hbm.at[pl.ds(row0, RPW)], idxv)

    hsplat = [jnp.full((L,), h, jnp.int32) for h in range(NUM_HEADS)]

    def body(g, c):
        r = g >> 4
        col = (g & 15) * L
        ivec = idxv[r, pl.ds(col, L)]
        vals = [
            plsc.load_gather(tabv, [ivec, hsplat[h]]) for h in range(NUM_HEADS)
        ]
        for h in range(NUM_HEADS):
            outv[h, r, pl.ds(col, L)] = vals[h]
        return c

    lax.fori_loop(0, GROUPS, body, 0)
    pltpu.sync_copy(outv, out_hbm.at[0, :, pl.ds(row0, RPW), :])


def kernel(relative_position_bias_table, relative_position_index):
    return _gather_bias(relative_position_bias_table, relative_position_index)


# speedup vs baseline: 8.3358x; 1.2602x over previous
import functools

import jax
import jax.numpy as jnp
from jax import lax
from jax.experimental import pallas as pl
from jax.experimental.pallas import tpu as pltpu
from jax.experimental.pallas import tpu_sc as plsc

NUM_HEADS = 16
T = 256
NC, NS, L = 2, 16, 16
NW = NC * NS
RPW = T // NW
GROUPS = RPW * T // L
TAB = 961 * NUM_HEADS

_mesh = plsc.VectorSubcoreMesh(core_axis_name="c", subcore_axis_name="s")


@functools.partial(
    pl.kernel,
    mesh=_mesh,
    compiler_params=pltpu.CompilerParams(
        needs_layout_passes=False, use_tc_tiling_on_sc=True
    ),
    out_type=jax.ShapeDtypeStruct((1, NUM_HEADS, T, T), jnp.float32),
    scratch_types=[
        pltpu.VMEM((TAB,), jnp.float32),
        pltpu.VMEM((RPW, T), jnp.int32),
        pltpu.VMEM((NUM_HEADS, RPW, T), jnp.float32),
    ],
)
def _gather_bias(tab_hbm, idx_hbm, out_hbm, tabv, idxv, outv):
    wid = lax.axis_index("s") * NC + lax.axis_index("c")
    row0 = wid * RPW
    pltpu.sync_copy(tab_hbm, tabv)
    pltpu.sync_copy(idx_hbm.at[pl.ds(row0, RPW)], idxv)

    def body(g, c):
        r = g >> 4
        col = (g & 15) * L
        ivec = idxv[r, pl.ds(col, L)]
        fbase = ivec * NUM_HEADS
        vals = [
            plsc.load_gather(tabv, [fbase + h]) for h in range(NUM_HEADS)
        ]
        for h in range(NUM_HEADS):
            outv[h, r, pl.ds(col, L)] = vals[h]
        return c

    lax.fori_loop(0, GROUPS, body, 0)
    pltpu.sync_copy(outv, out_hbm.at[0, :, pl.ds(row0, RPW), :])


def kernel(relative_position_bias_table, relative_position_index):
    tab = relative_position_bias_table.reshape(-1)
    return _gather_bias(tab, relative_position_index)


# speedup vs baseline: 9.1020x; 1.0919x over previous
import functools

import jax
import jax.numpy as jnp
from jax import lax
from jax.experimental import pallas as pl
from jax.experimental.pallas import tpu as pltpu
from jax.experimental.pallas import tpu_sc as plsc

NUM_HEADS = 16
T = 256
NC, NS, L = 2, 16, 16
NW = NC * NS
RPW = T // NW
GROUPS = RPW * T // L
TAB = 961 * NUM_HEADS

_mesh = plsc.VectorSubcoreMesh(core_axis_name="c", subcore_axis_name="s")


@functools.partial(
    pl.kernel,
    mesh=_mesh,
    compiler_params=pltpu.CompilerParams(
        needs_layout_passes=False, use_tc_tiling_on_sc=True
    ),
    out_type=jax.ShapeDtypeStruct((1, NUM_HEADS, T, T), jnp.float32),
    scratch_types=[
        pltpu.VMEM((TAB,), jnp.float32),
        pltpu.VMEM((RPW, T), jnp.int32),
        pltpu.VMEM((NUM_HEADS, RPW, T), jnp.float32),
    ],
)
def _gather_bias(tab_hbm, idx_hbm, out_hbm, tabv, idxv, outv):
    wid = lax.axis_index("s") * NC + lax.axis_index("c")
    row0 = wid * RPW
    pltpu.sync_copy(tab_hbm, tabv)
    pltpu.sync_copy(idx_hbm.at[pl.ds(row0, RPW)], idxv)

    def body(g, c):
        r = g >> 4
        col = (g & 15) * L
        ivec = idxv[r, pl.ds(col, L)]
        vals = [
            plsc.load_gather(tabv, [ivec + h * 961]) for h in range(NUM_HEADS)
        ]
        for h in range(NUM_HEADS):
            outv[h, r, pl.ds(col, L)] = vals[h]
        return c

    lax.fori_loop(0, GROUPS, body, 0)
    pltpu.sync_copy(outv, out_hbm.at[0, :, pl.ds(row0, RPW), :])


def kernel(relative_position_bias_table, relative_position_index):
    tab = relative_position_bias_table.T.reshape(-1)
    return _gather_bias(tab, relative_position_index)


# speedup vs baseline: 9.2041x; 1.0112x over previous
import functools

import jax
import jax.numpy as jnp
from jax import lax
from jax.experimental import pallas as pl
from jax.experimental.pallas import tpu as pltpu
from jax.experimental.pallas import tpu_sc as plsc

NUM_HEADS = 16
T = 256
NC, NS, L = 2, 16, 16
NW = NC * NS
RPW = T // NW
GROUPS = RPW * T // L
TAB = 961 * NUM_HEADS

_mesh = plsc.VectorSubcoreMesh(core_axis_name="c", subcore_axis_name="s")


@functools.partial(
    pl.kernel,
    mesh=_mesh,
    compiler_params=pltpu.CompilerParams(
        needs_layout_passes=False, use_tc_tiling_on_sc=True
    ),
    out_type=jax.ShapeDtypeStruct((1, NUM_HEADS, T, T), jnp.float32),
    scratch_types=[
        pltpu.VMEM((TAB,), jnp.float32),
        pltpu.VMEM((RPW, T), jnp.int32),
        pltpu.VMEM((NUM_HEADS, RPW, T), jnp.float32),
        pltpu.SemaphoreType.DMA,
        pltpu.SemaphoreType.DMA,
        pltpu.SemaphoreType.DMA,
        pltpu.SemaphoreType.DMA,
    ],
)
def _gather_bias(tab_hbm, idx_hbm, out_hbm, tabv, idxv, outv, st, si, so0, so1):
    wid = lax.axis_index("s") * NC + lax.axis_index("c")
    row0 = wid * RPW
    ct = pltpu.async_copy(tab_hbm, tabv, st)
    ci = pltpu.async_copy(idx_hbm.at[pl.ds(row0, RPW)], idxv, si)
    ci.wait()
    ct.wait()

    def half(c0):
        def body(g, c):
            r = g >> 3
            col = c0 + (g & 7) * L
            ivec = idxv[r, pl.ds(col, L)]
            vals = [
                plsc.load_gather(tabv, [ivec + h * 961])
                for h in range(NUM_HEADS)
            ]
            for h in range(NUM_HEADS):
                outv[h, r, pl.ds(col, L)] = vals[h]
            return c

        lax.fori_loop(0, GROUPS // 2, body, 0, unroll=2)

    half(0)
    co0 = pltpu.async_copy(
        outv.at[:, :, pl.ds(0, 128)],
        out_hbm.at[0, :, pl.ds(row0, RPW), pl.ds(0, 128)],
        so0,
    )
    half(128)
    co1 = pltpu.async_copy(
        outv.at[:, :, pl.ds(128, 128)],
        out_hbm.at[0, :, pl.ds(row0, RPW), pl.ds(128, 128)],
        so1,
    )
    co0.wait()
    co1.wait()


def kernel(relative_position_bias_table, relative_position_index):
    tab = relative_position_bias_table.T.reshape(-1)
    return _gather_bias(tab, relative_position_index)
